# Initial kernel scaffold; baseline (speedup 1.0000x reference)
import jax
import jax.numpy as jnp
from jax.experimental import pallas as pl


def kernel(variable_prediction, graph_map, batch_variable_map, batch_function_map, edge_feature):
    raise NotImplementedError("write your pallas kernel here")


# speedup vs baseline: 174.7481x; 174.7481x over previous
import functools

import jax
import jax.numpy as jnp
from jax import lax
from jax.experimental import pallas as pl
from jax.experimental.pallas import tpu as pltpu
from jax.experimental.pallas import tpu_sc as plsc

NC = 2
NS = 16
NW = NC * NS
LANES = 16
CHUNK_ROWS = 16
CHUNK = CHUNK_ROWS * 128


def _sc_edge_kernel(V, E, F_pad):
  n_chunks = E // CHUNK
  base_t, extra = divmod(n_chunks, NW)
  acc_slice = F_pad // NS
  mesh = plsc.VectorSubcoreMesh(core_axis_name="c", subcore_axis_name="s")

  def body(vp_hbm, gm_hbm, ef_hbm, out_hbm,
           vp_v, src_v, dst_v, ef_v, vals_v, zeros_v, acc_sh):
    cid = lax.axis_index("c")
    sid = lax.axis_index("s")
    wid = sid * NC + cid

    pltpu.sync_copy(vp_hbm, vp_v)

    def zero_body(i, _):
      zeros_v[pl.ds(i * LANES, LANES)] = jnp.zeros((LANES,), jnp.float32)
      return ()
    lax.fori_loop(0, acc_slice // LANES, zero_body, ())
    pltpu.sync_copy(zeros_v, acc_sh.at[pl.ds(sid * acc_slice, acc_slice)])
    plsc.subcore_barrier()

    n_mine = base_t + jnp.where(wid < extra, 1, 0)

    def chunk_body(t, _):
      c = wid + t * NW
      pltpu.sync_copy(gm_hbm.at[0, pl.ds(c * CHUNK, CHUNK)], src_v)
      pltpu.sync_copy(gm_hbm.at[1, pl.ds(c * CHUNK, CHUNK)], dst_v)
      pltpu.sync_copy(ef_hbm.at[pl.ds(c * CHUNK, CHUNK)], ef_v)

      def grp_body(k, _):
        s16 = src_v[pl.ds(k * LANES, LANES)]
        e16 = ef_v[pl.ds(k * LANES, LANES)]
        v16 = plsc.load_gather(vp_v, [s16])
        t16 = e16 * v16 + (1.0 - e16) / 2.0
        vals_v[pl.ds(k * LANES, LANES)] = jnp.where(
            t16 > 0.5, 1.0, 0.0).astype(jnp.float32)
        return ()
      lax.fori_loop(0, CHUNK // LANES, grp_body, ())

      pltpu.sync_copy(vals_v, acc_sh.at[dst_v], add=True)
      return ()
    lax.fori_loop(0, n_mine, chunk_body, ())

    plsc.subcore_barrier()
    pltpu.sync_copy(acc_sh.at[pl.ds(sid * acc_slice, acc_slice)],
                    out_hbm.at[cid, pl.ds(sid * acc_slice, acc_slice)])

  return pl.kernel(
      body,
      out_type=jax.ShapeDtypeStruct((NC, F_pad), jnp.float32),
      mesh=mesh,
      compiler_params=pltpu.CompilerParams(needs_layout_passes=False),
      scratch_types=[
          pltpu.VMEM((V,), jnp.float32),
          pltpu.VMEM((CHUNK,), jnp.int32),
          pltpu.VMEM((CHUNK,), jnp.int32),
          pltpu.VMEM((CHUNK,), jnp.float32),
          pltpu.VMEM((CHUNK,), jnp.float32),
          pltpu.VMEM((F_pad // NS,), jnp.float32),
          pltpu.VMEM_SHARED((F_pad,), jnp.float32),
      ],
  )


def _tc_epilogue_kernel(F_pad, B):
  rows = F_pad // 128
  block_rows = next(b for b in (128, 112, 96, 80, 64, 56, 48, 40, 32, 24, 16, 8)
                    if rows % b == 0)
  grid = rows // block_rows

  def body(part_ref, bfm_ref, cv_ref, sat_ref, unsat_ref, acc_bv, acc_ms):
    g = pl.program_id(0)

    @pl.when(g == 0)
    def _():
      acc_bv[...] = jnp.zeros((B, 128), jnp.float32)
      acc_ms[...] = jnp.zeros((B, 128), jnp.float32)

    s = part_ref[0] + part_ref[1]
    cv = (s > 0.0).astype(jnp.float32)
    cv_ref[...] = cv
    b = bfm_ref[...]
    for k in range(B):
      m = b == k
      acc_bv[k:k + 1, :] += jnp.sum(jnp.where(m, cv, 0.0), axis=0,
                                    keepdims=True)
      acc_ms[k:k + 1, :] += jnp.sum(m.astype(jnp.float32), axis=0,
                                    keepdims=True)

    @pl.when(g == grid - 1)
    def _():
      bv = jnp.sum(acc_bv[...], axis=1, keepdims=True)
      ms = jnp.sum(acc_ms[...], axis=1, keepdims=True)
      sat_ref[...] = jnp.broadcast_to(
          (ms == bv).astype(jnp.float32), (B, 128))
      unsat_ref[...] = jnp.broadcast_to(ms - bv, (B, 128))

  return pl.pallas_call(
      body,
      grid=(grid,),
      in_specs=[
          pl.BlockSpec((2, block_rows, 128), lambda g: (0, g, 0)),
          pl.BlockSpec((block_rows, 128), lambda g: (g, 0)),
      ],
      out_specs=[
          pl.BlockSpec((block_rows, 128), lambda g: (g, 0)),
          pl.BlockSpec((B, 128), lambda g: (0, 0)),
          pl.BlockSpec((B, 128), lambda g: (0, 0)),
      ],
      out_shape=[
          jax.ShapeDtypeStruct((rows, 128), jnp.float32),
          jax.ShapeDtypeStruct((B, 128), jnp.float32),
          jax.ShapeDtypeStruct((B, 128), jnp.float32),
      ],
      scratch_shapes=[
          pltpu.VMEM((B, 128), jnp.float32),
          pltpu.VMEM((B, 128), jnp.float32),
      ],
  )


@jax.jit
def kernel(variable_prediction, graph_map, batch_variable_map,
           batch_function_map, edge_feature):
  V = variable_prediction.shape[0]
  E = graph_map.shape[1]
  F = batch_function_map.shape[0]
  B = 16
  F_pad = ((F + 2047) // 2048) * 2048

  V_pad = ((V + 127) // 128) * 128
  vp = jnp.concatenate(
      [variable_prediction.reshape(V),
       jnp.zeros((V_pad - V,), jnp.float32)])
  ef1 = edge_feature.reshape(E)

  partial = _sc_edge_kernel(V_pad, E, F_pad)(vp, graph_map, ef1)

  bfm_pad = jnp.concatenate(
      [batch_function_map,
       jnp.full((F_pad - F,), B, jnp.int32)]).reshape(F_pad // 128, 128)
  part3 = partial.reshape(2, F_pad // 128, 128)

  cv, sat, unsat = _tc_epilogue_kernel(F_pad, B)(part3, bfm_pad)

  clause_values = cv.reshape(F_pad)[:F][:, None]
  sat_flag = sat[:, :1]
  unsat_count = unsat[:, :1]
  return (sat_flag, unsat_count, clause_values)


# speedup vs baseline: 355.2050x; 2.0327x over previous
import functools

import jax
import jax.numpy as jnp
from jax import lax
from jax.experimental import pallas as pl
from jax.experimental.pallas import tpu as pltpu
from jax.experimental.pallas import tpu_sc as plsc

NC = 2
NS = 16
NW = NC * NS
LANES = 16
CHUNK_ROWS = 16
CHUNK = CHUNK_ROWS * 128


NBUF = 3


def _sc_edge_kernel(V, E, F_pad):
  n_chunks = E // CHUNK
  assert n_chunks * CHUNK == E
  base_t, extra = divmod(n_chunks, NW)
  assert base_t >= NBUF
  max_n = base_t + (1 if extra else 0)
  n_groups = (max_n + NBUF - 1) // NBUF
  acc_slice = F_pad // NS
  mesh = plsc.VectorSubcoreMesh(core_axis_name="c", subcore_axis_name="s")

  def body(vp_hbm, gm_hbm, ef_hbm, out_hbm,
           vp_v, src_v, dst_v, ef_v, vals_v, acc_sh,
           in_sems, sc_sems):
    cid = lax.axis_index("c")
    sid = lax.axis_index("s")
    wid = sid * NC + cid

    pltpu.sync_copy(vp_hbm, vp_v)

    def zero_body(i, _):
      vals_v[0][pl.ds(i * LANES, LANES)] = jnp.zeros((LANES,), jnp.float32)
      return ()
    lax.fori_loop(0, CHUNK // LANES, zero_body, ())
    base = sid * acc_slice
    n_full, rem = divmod(acc_slice, CHUNK)
    for i in range(n_full):
      pltpu.sync_copy(vals_v[0], acc_sh.at[pl.ds(base + i * CHUNK, CHUNK)])
    if rem:
      pltpu.sync_copy(vals_v[0].at[pl.ds(0, rem)],
                      acc_sh.at[pl.ds(base + n_full * CHUNK, rem)])
    plsc.subcore_barrier()

    n_mine = base_t + jnp.where(wid < extra, 1, 0)

    def start_inputs(t, b):
      c = wid + t * NW
      pltpu.async_copy(gm_hbm.at[0, pl.ds(c * CHUNK, CHUNK)], src_v[b],
                       in_sems[b])
      pltpu.async_copy(gm_hbm.at[1, pl.ds(c * CHUNK, CHUNK)], dst_v[b],
                       in_sems[b])
      pltpu.async_copy(ef_hbm.at[pl.ds(c * CHUNK, CHUNK)], ef_v[b],
                       in_sems[b])

    def wait_inputs(b):
      pltpu.make_async_copy(gm_hbm.at[0, pl.ds(0, CHUNK)], src_v[b],
                            in_sems[b]).wait()
      pltpu.make_async_copy(gm_hbm.at[1, pl.ds(0, CHUNK)], dst_v[b],
                            in_sems[b]).wait()
      pltpu.make_async_copy(ef_hbm.at[pl.ds(0, CHUNK)], ef_v[b],
                            in_sems[b]).wait()

    def wait_scatter(b):
      pltpu.make_async_copy(vals_v[b], acc_sh.at[dst_v[b]],
                            sc_sems[b]).wait()

    for b in range(NBUF - 1):
      @pl.when(b < n_mine)
      def _(b=b):
        start_inputs(b, b)

    def group_body(g, _):
      for b in range(NBUF):
        t = g * NBUF + b

        @pl.when(t < n_mine)
        def _(t=t, b=b):
          wait_inputs(b)

          def grp_body(k, _):
            s16 = src_v[b][pl.ds(k * LANES, LANES)]
            e16 = ef_v[b][pl.ds(k * LANES, LANES)]
            v16 = plsc.load_gather(vp_v, [s16])
            t16 = e16 * v16 + (1.0 - e16) * 0.5
            vals_v[b][pl.ds(k * LANES, LANES)] = jnp.where(
                t16 > 0.5, 1.0, 0.0).astype(jnp.float32)
            return ()
          lax.fori_loop(0, CHUNK // LANES, grp_body, (), unroll=8)

          pltpu.async_copy(vals_v[b], acc_sh.at[dst_v[b]], sc_sems[b],
                           add=True)

          @pl.when(t + NBUF - 1 < n_mine)
          def _():
            b2 = (b + NBUF - 1) % NBUF

            @pl.when(t >= 1)
            def _():
              wait_scatter(b2)
            start_inputs(t + NBUF - 1, b2)
      return ()
    lax.fori_loop(0, n_groups, group_body, ())

    for b in range(NBUF):
      wait_scatter(b)

    plsc.subcore_barrier()
    pltpu.sync_copy(acc_sh.at[pl.ds(sid * acc_slice, acc_slice)],
                    out_hbm.at[cid, pl.ds(sid * acc_slice, acc_slice)])

  return pl.kernel(
      body,
      out_type=jax.ShapeDtypeStruct((NC, F_pad), jnp.float32),
      mesh=mesh,
      compiler_params=pltpu.CompilerParams(needs_layout_passes=False),
      scratch_types=[
          pltpu.VMEM((V,), jnp.float32),
          [pltpu.VMEM((CHUNK,), jnp.int32) for _ in range(NBUF)],
          [pltpu.VMEM((CHUNK,), jnp.int32) for _ in range(NBUF)],
          [pltpu.VMEM((CHUNK,), jnp.float32) for _ in range(NBUF)],
          [pltpu.VMEM((CHUNK,), jnp.float32) for _ in range(NBUF)],
          pltpu.VMEM_SHARED((F_pad,), jnp.float32),
          [pltpu.SemaphoreType.DMA for _ in range(NBUF)],
          [pltpu.SemaphoreType.DMA for _ in range(NBUF)],
      ],
  )


def _tc_epilogue_kernel(F_pad, B):
  rows = F_pad // 128
  block_rows = next(b for b in (128, 112, 96, 80, 64, 56, 48, 40, 32, 24, 16, 8)
                    if rows % b == 0)
  grid = rows // block_rows

  def body(part_ref, bfm_ref, cv_ref, sat_ref, unsat_ref, acc_bv, acc_ms):
    g = pl.program_id(0)

    @pl.when(g == 0)
    def _():
      acc_bv[...] = jnp.zeros((B, 128), jnp.float32)
      acc_ms[...] = jnp.zeros((B, 128), jnp.float32)

    s = part_ref[0] + part_ref[1]
    cv = (s > 0.0).astype(jnp.float32)
    cv_ref[...] = cv
    b = bfm_ref[...]
    for k in range(B):
      m = b == k
      acc_bv[k:k + 1, :] += jnp.sum(jnp.where(m, cv, 0.0), axis=0,
                                    keepdims=True)
      acc_ms[k:k + 1, :] += jnp.sum(m.astype(jnp.float32), axis=0,
                                    keepdims=True)

    @pl.when(g == grid - 1)
    def _():
      bv = jnp.sum(acc_bv[...], axis=1, keepdims=True)
      ms = jnp.sum(acc_ms[...], axis=1, keepdims=True)
      sat_ref[...] = jnp.broadcast_to(
          (ms == bv).astype(jnp.float32), (B, 128))
      unsat_ref[...] = jnp.broadcast_to(ms - bv, (B, 128))

  return pl.pallas_call(
      body,
      grid=(grid,),
      in_specs=[
          pl.BlockSpec((2, block_rows, 128), lambda g: (0, g, 0)),
          pl.BlockSpec((block_rows, 128), lambda g: (g, 0)),
      ],
      out_specs=[
          pl.BlockSpec((block_rows, 128), lambda g: (g, 0)),
          pl.BlockSpec((B, 128), lambda g: (0, 0)),
          pl.BlockSpec((B, 128), lambda g: (0, 0)),
      ],
      out_shape=[
          jax.ShapeDtypeStruct((rows, 128), jnp.float32),
          jax.ShapeDtypeStruct((B, 128), jnp.float32),
          jax.ShapeDtypeStruct((B, 128), jnp.float32),
      ],
      scratch_shapes=[
          pltpu.VMEM((B, 128), jnp.float32),
          pltpu.VMEM((B, 128), jnp.float32),
      ],
  )


@jax.jit
def kernel(variable_prediction, graph_map, batch_variable_map,
           batch_function_map, edge_feature):
  V = variable_prediction.shape[0]
  E = graph_map.shape[1]
  F = batch_function_map.shape[0]
  B = 16
  F_pad = ((F + 2047) // 2048) * 2048

  V_pad = ((V + 127) // 128) * 128
  vp = jnp.concatenate(
      [variable_prediction.reshape(V),
       jnp.zeros((V_pad - V,), jnp.float32)])
  ef1 = edge_feature.reshape(E)

  partial = _sc_edge_kernel(V_pad, E, F_pad)(vp, graph_map, ef1)

  bfm_pad = jnp.concatenate(
      [batch_function_map,
       jnp.full((F_pad - F,), B, jnp.int32)]).reshape(F_pad // 128, 128)
  part3 = partial.reshape(2, F_pad // 128, 128)

  cv, sat, unsat = _tc_epilogue_kernel(F_pad, B)(part3, bfm_pad)

  clause_values = cv.reshape(F_pad)[:F][:, None]
  sat_flag = sat[:, :1]
  unsat_count = unsat[:, :1]
  return (sat_flag, unsat_count, clause_values)
